# baseline (speedup 1.0000x reference)
import functools

import jax
import jax.numpy as jnp
from jax import lax
from jax.experimental import pallas as pl
from jax.experimental.pallas import tpu as pltpu
from jax.experimental.pallas import tpu_sc as plsc

NH = 64
RL = 2
RH = 32
IC = 3
K = 1024
D = 64
CC = 0.25
B = 8
HW = 224
HQ = 56
NPIX = HQ * HQ
NTOK = B * NPIX

F32 = jnp.float32

_PHR = ((1, 0), (0, 1), (1, 1), (0, 2))

_TAPS2 = (
    ((0, 1, 0), (2, 0, 1)),
    ((1, 0, 1), (3, 1, 1)),
    ((0, 0, 1), (2, 1, 1)),
    ((1, 1, 1), (3, 0, 2)),
)

_TAPS1 = (((0, 0), (2, 1)), ((1, 1), (3, 2)))


def _dot(a, b):
    return lax.dot_general(a, b, (((1,), (0,)), ((), ())),
                           preferred_element_type=F32)


def _enc_body(xim_ref, w1_ref, b1_ref, w2_ref, b2_ref, w3_ref, b3_ref,
              er1_ref, eb1_ref, er2_ref, eb2_ref, pv_ref, pvb_ref, cbt_ref,
              csq_ref, idx_ref, mind_ref, ph1_ref, hp_ref, tp_ref):
    ph1_ref[...] = jnp.zeros((58, 58, 128), F32)
    ys = []
    for ph in range(4):
        xi = xim_ref[0, :, :, ph * 48:(ph + 1) * 48].reshape(NPIX, 48)
        ys.append(jnp.maximum(_dot(xi, w1_ref[ph]) + b1_ref[...], 0.0))
    y = jnp.concatenate(ys, axis=1)
    ph1_ref[1:57, 1:57, :] = y.reshape(HQ, HQ, 128)

    acc = None
    for orr in range(3):
        for oc in range(3):
            sl = ph1_ref[orr:orr + HQ, oc:oc + HQ, :].reshape(NPIX, 128)
            t = _dot(sl, w2_ref[orr * 3 + oc])
            acc = t if acc is None else acc + t
    h2 = jnp.maximum(acc + b2_ref[...], 0.0)
    hp_ref[...] = jnp.zeros((58, 58, NH), F32)
    hp_ref[1:57, 1:57, :] = h2.reshape(HQ, HQ, NH)

    h = None
    for di in range(3):
        for dj in range(3):
            sl = hp_ref[di:di + HQ, dj:dj + HQ, :].reshape(NPIX, NH)
            t = _dot(sl, w3_ref[di * 3 + dj])
            h = t if h is None else h + t
    h = h + b3_ref[...]

    for l in range(RL):
        tr = jnp.maximum(h, 0.0)
        tp_ref[...] = jnp.zeros((58, 58, NH), F32)
        tp_ref[1:57, 1:57, :] = tr.reshape(HQ, HQ, NH)
        t1 = None
        for di in range(3):
            for dj in range(3):
                sl = tp_ref[di:di + HQ, dj:dj + HQ, :].reshape(NPIX, NH)
                u = _dot(sl, er1_ref[l, di * 3 + dj])
                t1 = u if t1 is None else t1 + u
        t1 = jnp.maximum(t1 + eb1_ref[l], 0.0)
        t2 = _dot(t1, er2_ref[l]) + eb2_ref[l]
        h = h + t2
    h = jnp.maximum(h, 0.0)

    z = _dot(h, pv_ref[...]) + pvb_ref[...]

    cbt = cbt_ref[...]
    TT = 784
    for t in range(NPIX // TT):
        zt = z[t * TT:(t + 1) * TT, :]
        zsq = jnp.sum(zt * zt, axis=1, keepdims=True)
        dist = zsq - 2.0 * _dot(zt, cbt) + csq_ref[...]
        m = jnp.min(dist, axis=1, keepdims=True)
        ii = lax.broadcasted_iota(jnp.int32, (TT, K), 1)
        sel = jnp.where(dist <= m, ii, jnp.int32(1 << 30))
        idx_ref[0, 0, t * TT:(t + 1) * TT] = jnp.min(sel, axis=1)
        mind_ref[0, 0, t * TT:(t + 1) * TT] = m[:, 0]


def _dec_body(q_ref, mind_ref, cnt_ref, d1_ref, d1b_ref,
              dr1_ref, db1_ref, dr2_ref, db2_ref,
              t1_ref, t1b_ref, t2_ref, t2b_ref,
              xr_ref, vq_ref, pp_ref, qp_ref, tp_ref, phT_ref):
    b = pl.program_id(0)

    qp_ref[...] = jnp.zeros((58, 58, NH), F32)
    qp_ref[1:57, 1:57, :] = q_ref[0, :, :D].reshape(HQ, HQ, NH)
    h = None
    for di in range(3):
        for dj in range(3):
            sl = qp_ref[di:di + HQ, dj:dj + HQ, :].reshape(NPIX, NH)
            t = _dot(sl, d1_ref[di * 3 + dj])
            h = t if h is None else h + t
    h = h + d1b_ref[...]

    for l in range(RL):
        tr = jnp.maximum(h, 0.0)
        tp_ref[...] = jnp.zeros((58, 58, NH), F32)
        tp_ref[1:57, 1:57, :] = tr.reshape(HQ, HQ, NH)
        t1 = None
        for di in range(3):
            for dj in range(3):
                sl = tp_ref[di:di + HQ, dj:dj + HQ, :].reshape(NPIX, NH)
                u = _dot(sl, dr1_ref[l, di * 3 + dj])
                t1 = u if t1 is None else t1 + u
        t1 = jnp.maximum(t1 + db1_ref[l], 0.0)
        t2 = _dot(t1, dr2_ref[l]) + db2_ref[l]
        h = h + t2
    h = jnp.maximum(h, 0.0)

    qp_ref[...] = jnp.zeros((58, 58, NH), F32)
    qp_ref[1:57, 1:57, :] = h.reshape(HQ, HQ, NH)
    phT_ref[...] = jnp.zeros((58, 58, 128), F32)
    acc = None
    for orr in range(3):
        for oc in range(3):
            sl = qp_ref[orr:orr + HQ, oc:oc + HQ, :].reshape(NPIX, NH)
            u = _dot(sl, t1_ref[orr * 3 + oc])
            acc = u if acc is None else acc + u
    y = jnp.maximum(acc + t1b_ref[...], 0.0)
    phT_ref[1:57, 1:57, :] = y.reshape(HQ, HQ, 128)

    acc = None
    for orr in range(3):
        for oc in range(3):
            sl = phT_ref[orr:orr + HQ, oc:oc + HQ, :].reshape(NPIX, 128)
            u = _dot(sl, t2_ref[orr * 3 + oc])
            acc = u if acc is None else acc + u
    y = acc + t2b_ref[...]
    xr_ref[0] = y.reshape(HQ, HQ, 48)

    s = jnp.sum(mind_ref[0], axis=1, keepdims=True)

    @pl.when(b == 0)
    def _():
        vq_ref[...] = jnp.zeros((1, 1), F32)
        tot = jnp.sum(cnt_ref[...], axis=0, keepdims=True)
        p = tot / jnp.float32(NTOK)
        ent = jnp.sum(p * jnp.log(p + 1e-10), axis=1, keepdims=True)
        pp_ref[...] = jnp.exp(-ent)

    vq_ref[...] = vq_ref[...] + s * ((1.0 + CC) / (NTOK * D))


def _sc_gather_hist(cb, idx):
    info = plsc.get_sparse_core_info()
    NC, NS = info.num_cores, info.num_subcores
    NW = NC * NS
    bpw = NTOK // NW
    DP = 2 * D
    mesh = plsc.VectorSubcoreMesh(core_axis_name="c", subcore_axis_name="s")

    @functools.partial(
        pl.kernel, mesh=mesh,
        out_type=(jax.ShapeDtypeStruct((NTOK, DP), F32),
                  jax.ShapeDtypeStruct((NW, K), F32)),
        scratch_types=[
            pltpu.VMEM((bpw,), jnp.int32),
            pltpu.VMEM((bpw, DP), F32),
            pltpu.VMEM((bpw,), F32),
            pltpu.VMEM((bpw,), jnp.int32),
            pltpu.VMEM((K,), F32),
            pltpu.VMEM_SHARED((NS * K,), F32),
            pltpu.SemaphoreType.DMA,
        ],
    )
    def k(cb_hbm, idx_hbm, q_hbm, cnt_hbm, idx_v, rows_v, ones_v, off_v,
          zer_v, shared, sem):
        cid = lax.axis_index("c")
        sid = lax.axis_index("s")
        wid = sid * NC + cid
        base = wid * bpw
        pltpu.sync_copy(idx_hbm.at[pl.ds(base, bpw)], idx_v)
        pltpu.async_copy(cb_hbm.at[idx_v], rows_v, sem).wait()
        pltpu.sync_copy(rows_v, q_hbm.at[pl.ds(base, bpw)])
        for i in range(K // 16):
            zer_v[pl.ds(i * 16, 16)] = jnp.zeros((16,), F32)
        pltpu.sync_copy(zer_v, shared.at[pl.ds(sid * K, K)])
        for i in range(bpw // 16):
            ones_v[pl.ds(i * 16, 16)] = jnp.ones((16,), F32)
            off_v[pl.ds(i * 16, 16)] = (idx_v[pl.ds(i * 16, 16)]
                                        + jnp.int32(1) * sid * K)
        pltpu.sync_copy(ones_v, shared.at[off_v], add=True)
        pltpu.sync_copy(shared.at[pl.ds(sid * K, K)], cnt_hbm.at[wid])

    return k(cb, idx)


def _enc_call(xim, wts, interpret=False):
    full = lambda a: pl.BlockSpec(a.shape, lambda b: (0,) * a.ndim)
    return pl.pallas_call(
        _enc_body,
        grid=(B,),
        in_specs=[pl.BlockSpec((1, HQ, HQ, 192), lambda b: (b, 0, 0, 0))]
                 + [full(w) for w in wts],
        out_specs=[pl.BlockSpec((1, 1, NPIX), lambda b: (b, 0, 0)),
                   pl.BlockSpec((1, 1, NPIX), lambda b: (b, 0, 0))],
        out_shape=[jax.ShapeDtypeStruct((B, 1, NPIX), jnp.int32),
                   jax.ShapeDtypeStruct((B, 1, NPIX), F32)],
        scratch_shapes=[pltpu.VMEM((58, 58, 128), F32),
                        pltpu.VMEM((58, 58, NH), F32),
                        pltpu.VMEM((58, 58, NH), F32)],
        interpret=interpret,
    )(xim, *wts)


def _dec_call(q8, mind, cnt, wts, interpret=False):
    full = lambda a: pl.BlockSpec(a.shape, lambda b: (0,) * a.ndim)
    return pl.pallas_call(
        _dec_body,
        grid=(B,),
        in_specs=[pl.BlockSpec((1, NPIX, 2 * D), lambda b: (b, 0, 0)),
                  pl.BlockSpec((1, 1, NPIX), lambda b: (b, 0, 0)),
                  full(cnt)] + [full(w) for w in wts],
        out_specs=[pl.BlockSpec((1, HQ, HQ, 48), lambda b: (b, 0, 0, 0)),
                   pl.BlockSpec((1, 1), lambda b: (0, 0)),
                   pl.BlockSpec((1, 1), lambda b: (0, 0))],
        out_shape=[jax.ShapeDtypeStruct((B, HQ, HQ, 48), F32),
                   jax.ShapeDtypeStruct((1, 1), F32),
                   jax.ShapeDtypeStruct((1, 1), F32)],
        scratch_shapes=[pltpu.VMEM((58, 58, NH), F32),
                        pltpu.VMEM((58, 58, NH), F32),
                        pltpu.VMEM((58, 58, 128), F32)],
        interpret=interpret,
    )(q8, mind, cnt, *wts)


def _prep_xim(x):
    xh = jnp.transpose(x, (0, 2, 3, 1))
    xpad = jnp.pad(xh, ((0, 0), (1, 1), (1, 1), (0, 0)))
    blocks = []
    for a2 in range(2):
        for c2 in range(2):
            taps = []
            for kh in range(4):
                for kw in range(4):
                    r0 = 2 * a2 + kh
                    c0 = 2 * c2 + kw
                    sl = lax.slice(xpad, (0, r0, c0, 0),
                                   (B, r0 + 4 * (HQ - 1) + 1,
                                    c0 + 4 * (HQ - 1) + 1, IC),
                                   (1, 4, 4, 1))
                    taps.append(sl)
            blocks.append(jnp.concatenate(taps, axis=-1))
    return jnp.concatenate(blocks, axis=-1)


def _enc_wts(p):
    w1 = jnp.transpose(p['e1w'], (2, 3, 1, 0)).reshape(48, NH // 2)
    w1big = jnp.stack([w1] * 4)
    b1big = p['e1b'].reshape(1, NH // 2)

    w2big = jnp.zeros((9, 128, NH), F32)
    for kh in range(4):
        pr, orr = _PHR[kh]
        for kw in range(4):
            pc, oc = _PHR[kw]
            blk = (pr * 2 + pc) * 32
            w2big = w2big.at[orr * 3 + oc, blk:blk + 32, :].set(
                p['e2w'][:, :, kh, kw].T)

    er1 = jnp.stack([jnp.transpose(p[f'er{l}w1'], (2, 3, 1, 0))
                     .reshape(9, NH, RH) for l in range(RL)])
    eb1 = jnp.stack([p[f'er{l}b1'].reshape(1, RH) for l in range(RL)])
    er2 = jnp.stack([p[f'er{l}w2'][:, :, 0, 0].T for l in range(RL)])
    eb2 = jnp.stack([p[f'er{l}b2'].reshape(1, NH) for l in range(RL)])
    return [
        w1big, b1big, w2big, p['e2b'].reshape(1, NH),
        jnp.transpose(p['e3w'], (2, 3, 1, 0)).reshape(9, NH, NH),
        p['e3b'].reshape(1, NH),
        er1, eb1, er2, eb2,
        p['pvw'][:, :, 0, 0].T,
        p['pvb'].reshape(1, D),
        p['cb'].T,
        jnp.sum(p['cb'] ** 2, axis=1).reshape(1, K),
    ]


def _t2_maps():
    m = {}
    for t in range(4):
        for kh, psrc, off in _TAPS2[t]:
            m.setdefault((psrc, off), []).append((t, kh))
    return m


def _dec_wts(p):
    dr1 = jnp.stack([jnp.transpose(p[f'dr{l}w1'], (2, 3, 1, 0))
                     .reshape(9, NH, RH) for l in range(RL)])
    db1 = jnp.stack([p[f'dr{l}b1'].reshape(1, RH) for l in range(RL)])
    dr2 = jnp.stack([p[f'dr{l}w2'][:, :, 0, 0].T for l in range(RL)])
    db2 = jnp.stack([p[f'dr{l}b2'].reshape(1, NH) for l in range(RL)])

    rm1 = {}
    for r in range(2):
        for kh, off in _TAPS1[r]:
            rm1.setdefault(off, []).append((r, kh))
    t1big = jnp.zeros((9, NH, 128), F32)
    for orr in range(3):
        for oc in range(3):
            for r, kh in rm1[orr]:
                for s, kw in rm1[oc]:
                    blk = (r * 2 + s) * 32
                    t1big = t1big.at[orr * 3 + oc, :, blk:blk + 32].set(
                        jnp.transpose(p['dt1w'][:, :, kh, kw], (1, 0)))
    t1b = jnp.tile(p['dt1b'], 4).reshape(1, 128)

    m2 = _t2_maps()
    t2big = jnp.zeros((9, 128, 48), F32)
    for orr in range(3):
        for oc in range(3):
            for pr in range(2):
                if (pr, orr) not in m2:
                    continue
                for pc in range(2):
                    if (pc, oc) not in m2:
                        continue
                    row = (pr * 2 + pc) * 32
                    for tr, kh in m2[(pr, orr)]:
                        for tc, kw in m2[(pc, oc)]:
                            col = tr * 12 + tc * 3
                            t2big = t2big.at[
                                orr * 3 + oc, row:row + 32, col:col + 3].set(
                                jnp.transpose(p['dt2w'][:, :, kh, kw], (1, 0)))
    t2b = jnp.tile(p['dt2b'], 16).reshape(1, 48)

    return [
        jnp.transpose(p['d1w'], (2, 3, 1, 0)).reshape(9, D, NH),
        p['d1b'].reshape(1, NH),
        dr1, db1, dr2, db2,
        t1big, t1b, t2big, t2b,
    ]


def kernel(x, params):
    p = params
    xim = _prep_xim(x)
    idx8, mind = _enc_call(xim, _enc_wts(p))
    idx_flat = idx8.reshape(NTOK)
    cb128 = jnp.pad(p['cb'], ((0, 0), (0, D)))
    q, cnt = _sc_gather_hist(cb128, idx_flat)
    q8 = q.reshape(B, NPIX, 2 * D)
    xr, vq, pp = _dec_call(q8, mind, cnt, _dec_wts(p))
    xr6 = xr.reshape(B, HQ, HQ, 4, 4, IC)
    x_recon = jnp.transpose(xr6, (0, 5, 1, 3, 2, 4)).reshape(B, IC, HW, HW)
    return x_recon, vq[0, 0], pp[0, 0], idx_flat


# speedup vs baseline: 1.0160x; 1.0160x over previous
import functools

import jax
import jax.numpy as jnp
from jax import lax
from jax.experimental import pallas as pl
from jax.experimental.pallas import tpu as pltpu
from jax.experimental.pallas import tpu_sc as plsc

NH = 64
RL = 2
RH = 32
IC = 3
K = 1024
D = 64
CC = 0.25
B = 8
HW = 224
HQ = 56
NPIX = HQ * HQ
NTOK = B * NPIX

F32 = jnp.float32

_PHR = ((1, 0), (0, 1), (1, 1), (0, 2))

_TAPS2 = (
    ((0, 1, 0), (2, 0, 1)),
    ((1, 0, 1), (3, 1, 1)),
    ((0, 0, 1), (2, 1, 1)),
    ((1, 1, 1), (3, 0, 2)),
)

_TAPS1 = (((0, 0), (2, 1)), ((1, 1), (3, 2)))


def _dot(a, b):
    return lax.dot_general(a, b, (((1,), (0,)), ((), ())),
                           preferred_element_type=F32)


def _enc_body(xim_ref, w1_ref, b1_ref, w2_ref, b2_ref, w3_ref, b3_ref,
              er1_ref, eb1_ref, er2_ref, eb2_ref, pv_ref, pvb_ref, cbt_ref,
              csq_ref, idx_ref, mind_ref, ph1_ref, hp_ref, tp_ref):
    ph1_ref[...] = jnp.zeros((58, 58, 128), F32)
    ys = []
    for ph in range(4):
        xi = xim_ref[0, :, :, ph * 48:(ph + 1) * 48].reshape(NPIX, 48)
        ys.append(jnp.maximum(_dot(xi, w1_ref[ph]) + b1_ref[...], 0.0))
    y = jnp.concatenate(ys, axis=1)
    ph1_ref[1:57, 1:57, :] = y.reshape(HQ, HQ, 128)

    acc = None
    for orr in range(3):
        for oc in range(3):
            sl = ph1_ref[orr:orr + HQ, oc:oc + HQ, :].reshape(NPIX, 128)
            t = _dot(sl, w2_ref[orr * 3 + oc])
            acc = t if acc is None else acc + t
    h2 = jnp.maximum(acc + b2_ref[...], 0.0)
    hp_ref[...] = jnp.zeros((58, 58, NH), F32)
    hp_ref[1:57, 1:57, :] = h2.reshape(HQ, HQ, NH)

    h = None
    for di in range(3):
        for dj in range(3):
            sl = hp_ref[di:di + HQ, dj:dj + HQ, :].reshape(NPIX, NH)
            t = _dot(sl, w3_ref[di * 3 + dj])
            h = t if h is None else h + t
    h = h + b3_ref[...]

    for l in range(RL):
        tr = jnp.maximum(h, 0.0)
        tp_ref[...] = jnp.zeros((58, 58, NH), F32)
        tp_ref[1:57, 1:57, :] = tr.reshape(HQ, HQ, NH)
        t1 = None
        for di in range(3):
            for dj in range(3):
                sl = tp_ref[di:di + HQ, dj:dj + HQ, :].reshape(NPIX, NH)
                u = _dot(sl, er1_ref[l, di * 3 + dj])
                t1 = u if t1 is None else t1 + u
        t1 = jnp.maximum(t1 + eb1_ref[l], 0.0)
        t2 = _dot(t1, er2_ref[l]) + eb2_ref[l]
        h = h + t2
    h = jnp.maximum(h, 0.0)

    z = _dot(h, pv_ref[...]) + pvb_ref[...]

    cbt = cbt_ref[...]
    TT = 784
    for t in range(NPIX // TT):
        zt = z[t * TT:(t + 1) * TT, :]
        zsq = jnp.sum(zt * zt, axis=1, keepdims=True)
        dist = zsq - 2.0 * _dot(zt, cbt) + csq_ref[...]
        m = jnp.min(dist, axis=1, keepdims=True)
        ii = lax.broadcasted_iota(jnp.int32, (TT, K), 1)
        sel = jnp.where(dist <= m, ii, jnp.int32(1 << 30))
        idx_ref[0, 0, t * TT:(t + 1) * TT] = jnp.min(sel, axis=1)
        mind_ref[0, 0, t * TT:(t + 1) * TT] = m[:, 0]


def _dec_body(q_ref, mind_ref, idx_ref, d1_ref, d1b_ref,
              dr1_ref, db1_ref, dr2_ref, db2_ref,
              t1_ref, t1b_ref, t2_ref, t2b_ref,
              xr_ref, vq_ref, pp_ref, qp_ref, tp_ref, phT_ref, cnt_ref):
    b = pl.program_id(0)

    qp_ref[...] = jnp.zeros((58, 58, NH), F32)
    qp_ref[1:57, 1:57, :] = q_ref[0, :, :D].reshape(HQ, HQ, NH)
    h = None
    for di in range(3):
        for dj in range(3):
            sl = qp_ref[di:di + HQ, dj:dj + HQ, :].reshape(NPIX, NH)
            t = _dot(sl, d1_ref[di * 3 + dj])
            h = t if h is None else h + t
    h = h + d1b_ref[...]

    for l in range(RL):
        tr = jnp.maximum(h, 0.0)
        tp_ref[...] = jnp.zeros((58, 58, NH), F32)
        tp_ref[1:57, 1:57, :] = tr.reshape(HQ, HQ, NH)
        t1 = None
        for di in range(3):
            for dj in range(3):
                sl = tp_ref[di:di + HQ, dj:dj + HQ, :].reshape(NPIX, NH)
                u = _dot(sl, dr1_ref[l, di * 3 + dj])
                t1 = u if t1 is None else t1 + u
        t1 = jnp.maximum(t1 + db1_ref[l], 0.0)
        t2 = _dot(t1, dr2_ref[l]) + db2_ref[l]
        h = h + t2
    h = jnp.maximum(h, 0.0)

    qp_ref[...] = jnp.zeros((58, 58, NH), F32)
    qp_ref[1:57, 1:57, :] = h.reshape(HQ, HQ, NH)
    phT_ref[...] = jnp.zeros((58, 58, 128), F32)
    acc = None
    for orr in range(3):
        for oc in range(3):
            sl = qp_ref[orr:orr + HQ, oc:oc + HQ, :].reshape(NPIX, NH)
            u = _dot(sl, t1_ref[orr * 3 + oc])
            acc = u if acc is None else acc + u
    y = jnp.maximum(acc + t1b_ref[...], 0.0)
    phT_ref[1:57, 1:57, :] = y.reshape(HQ, HQ, 128)

    acc = None
    for orr in range(3):
        for oc in range(3):
            sl = phT_ref[orr:orr + HQ, oc:oc + HQ, :].reshape(NPIX, 128)
            u = _dot(sl, t2_ref[orr * 3 + oc])
            acc = u if acc is None else acc + u
    y = acc + t2b_ref[...]
    xr_ref[0] = y.reshape(HQ, HQ, 48)

    s = jnp.sum(mind_ref[0], axis=1, keepdims=True)
    ii = lax.broadcasted_iota(jnp.int32, (NPIX, K), 1)
    oh = jnp.where(ii == idx_ref[0].reshape(NPIX, 1), 1.0, 0.0)
    part = _dot(jnp.ones((1, NPIX), F32), oh)

    @pl.when(b == 0)
    def _():
        vq_ref[...] = jnp.zeros((1, 1), F32)
        cnt_ref[...] = jnp.zeros((1, K), F32)

    vq_ref[...] = vq_ref[...] + s * ((1.0 + CC) / (NTOK * D))
    cnt_ref[...] = cnt_ref[...] + part

    @pl.when(b == B - 1)
    def _():
        p = cnt_ref[...] / jnp.float32(NTOK)
        ent = jnp.sum(p * jnp.log(p + 1e-10), axis=1, keepdims=True)
        pp_ref[...] = jnp.exp(-ent)


def _sc_gather_hist(cb, idx):
    info = plsc.get_sparse_core_info()
    NC, NS = info.num_cores, info.num_subcores
    NW = NC * NS
    bpw = NTOK // NW
    DP = 2 * D
    mesh = plsc.VectorSubcoreMesh(core_axis_name="c", subcore_axis_name="s")

    @functools.partial(
        pl.kernel, mesh=mesh,
        out_type=jax.ShapeDtypeStruct((NTOK, DP), F32),
        scratch_types=[
            pltpu.VMEM((bpw,), jnp.int32),
            pltpu.VMEM((bpw, DP), F32),
            pltpu.VMEM_SHARED((K, DP), F32),
            pltpu.SemaphoreType.DMA,
        ],
    )
    def k(cb_hbm, idx_hbm, q_hbm, idx_v, rows_v, cbs, sem):
        cid = lax.axis_index("c")
        sid = lax.axis_index("s")
        wid = sid * NC + cid
        base = wid * bpw
        @pl.when(sid == 0)
        def _():
            pltpu.sync_copy(cb_hbm, cbs)

        pltpu.sync_copy(idx_hbm.at[pl.ds(base, bpw)], idx_v)
        plsc.subcore_barrier()
        pltpu.async_copy(cbs.at[idx_v], rows_v, sem).wait()
        pltpu.sync_copy(rows_v, q_hbm.at[pl.ds(base, bpw)])

    return k(cb, idx)


def _enc_call(xim, wts, interpret=False):
    full = lambda a: pl.BlockSpec(a.shape, lambda b: (0,) * a.ndim)
    return pl.pallas_call(
        _enc_body,
        grid=(B,),
        in_specs=[pl.BlockSpec((1, HQ, HQ, 192), lambda b: (b, 0, 0, 0))]
                 + [full(w) for w in wts],
        out_specs=[pl.BlockSpec((1, 1, NPIX), lambda b: (b, 0, 0)),
                   pl.BlockSpec((1, 1, NPIX), lambda b: (b, 0, 0))],
        out_shape=[jax.ShapeDtypeStruct((B, 1, NPIX), jnp.int32),
                   jax.ShapeDtypeStruct((B, 1, NPIX), F32)],
        scratch_shapes=[pltpu.VMEM((58, 58, 128), F32),
                        pltpu.VMEM((58, 58, NH), F32),
                        pltpu.VMEM((58, 58, NH), F32)],
        interpret=interpret,
    )(xim, *wts)


def _dec_call(q8, mind, idx8, wts, interpret=False):
    full = lambda a: pl.BlockSpec(a.shape, lambda b: (0,) * a.ndim)
    return pl.pallas_call(
        _dec_body,
        grid=(B,),
        in_specs=[pl.BlockSpec((1, NPIX, 2 * D), lambda b: (b, 0, 0)),
                  pl.BlockSpec((1, 1, NPIX), lambda b: (b, 0, 0)),
                  pl.BlockSpec((1, 1, NPIX), lambda b: (b, 0, 0))]
                 + [full(w) for w in wts],
        out_specs=[pl.BlockSpec((1, HQ, HQ, 48), lambda b: (b, 0, 0, 0)),
                   pl.BlockSpec((1, 1), lambda b: (0, 0)),
                   pl.BlockSpec((1, 1), lambda b: (0, 0))],
        out_shape=[jax.ShapeDtypeStruct((B, HQ, HQ, 48), F32),
                   jax.ShapeDtypeStruct((1, 1), F32),
                   jax.ShapeDtypeStruct((1, 1), F32)],
        scratch_shapes=[pltpu.VMEM((58, 58, NH), F32),
                        pltpu.VMEM((58, 58, NH), F32),
                        pltpu.VMEM((58, 58, 128), F32),
                        pltpu.VMEM((1, K), F32)],
        interpret=interpret,
    )(q8, mind, idx8, *wts)


def _prep_xim(x):
    xh = jnp.transpose(x, (0, 2, 3, 1))
    xpad = jnp.pad(xh, ((0, 0), (1, 1), (1, 1), (0, 0)))
    blocks = []
    for a2 in range(2):
        for c2 in range(2):
            taps = []
            for kh in range(4):
                for kw in range(4):
                    r0 = 2 * a2 + kh
                    c0 = 2 * c2 + kw
                    sl = lax.slice(xpad, (0, r0, c0, 0),
                                   (B, r0 + 4 * (HQ - 1) + 1,
                                    c0 + 4 * (HQ - 1) + 1, IC),
                                   (1, 4, 4, 1))
                    taps.append(sl)
            blocks.append(jnp.concatenate(taps, axis=-1))
    return jnp.concatenate(blocks, axis=-1)


def _enc_wts(p):
    w1 = jnp.transpose(p['e1w'], (2, 3, 1, 0)).reshape(48, NH // 2)
    w1big = jnp.stack([w1] * 4)
    b1big = p['e1b'].reshape(1, NH // 2)

    w2big = jnp.zeros((9, 128, NH), F32)
    for kh in range(4):
        pr, orr = _PHR[kh]
        for kw in range(4):
            pc, oc = _PHR[kw]
            blk = (pr * 2 + pc) * 32
            w2big = w2big.at[orr * 3 + oc, blk:blk + 32, :].set(
                p['e2w'][:, :, kh, kw].T)

    er1 = jnp.stack([jnp.transpose(p[f'er{l}w1'], (2, 3, 1, 0))
                     .reshape(9, NH, RH) for l in range(RL)])
    eb1 = jnp.stack([p[f'er{l}b1'].reshape(1, RH) for l in range(RL)])
    er2 = jnp.stack([p[f'er{l}w2'][:, :, 0, 0].T for l in range(RL)])
    eb2 = jnp.stack([p[f'er{l}b2'].reshape(1, NH) for l in range(RL)])
    return [
        w1big, b1big, w2big, p['e2b'].reshape(1, NH),
        jnp.transpose(p['e3w'], (2, 3, 1, 0)).reshape(9, NH, NH),
        p['e3b'].reshape(1, NH),
        er1, eb1, er2, eb2,
        p['pvw'][:, :, 0, 0].T,
        p['pvb'].reshape(1, D),
        p['cb'].T,
        jnp.sum(p['cb'] ** 2, axis=1).reshape(1, K),
    ]


def _t2_maps():
    m = {}
    for t in range(4):
        for kh, psrc, off in _TAPS2[t]:
            m.setdefault((psrc, off), []).append((t, kh))
    return m


def _dec_wts(p):
    dr1 = jnp.stack([jnp.transpose(p[f'dr{l}w1'], (2, 3, 1, 0))
                     .reshape(9, NH, RH) for l in range(RL)])
    db1 = jnp.stack([p[f'dr{l}b1'].reshape(1, RH) for l in range(RL)])
    dr2 = jnp.stack([p[f'dr{l}w2'][:, :, 0, 0].T for l in range(RL)])
    db2 = jnp.stack([p[f'dr{l}b2'].reshape(1, NH) for l in range(RL)])

    rm1 = {}
    for r in range(2):
        for kh, off in _TAPS1[r]:
            rm1.setdefault(off, []).append((r, kh))
    t1big = jnp.zeros((9, NH, 128), F32)
    for orr in range(3):
        for oc in range(3):
            for r, kh in rm1[orr]:
                for s, kw in rm1[oc]:
                    blk = (r * 2 + s) * 32
                    t1big = t1big.at[orr * 3 + oc, :, blk:blk + 32].set(
                        jnp.transpose(p['dt1w'][:, :, kh, kw], (1, 0)))
    t1b = jnp.tile(p['dt1b'], 4).reshape(1, 128)

    m2 = _t2_maps()
    t2big = jnp.zeros((9, 128, 48), F32)
    for orr in range(3):
        for oc in range(3):
            for pr in range(2):
                if (pr, orr) not in m2:
                    continue
                for pc in range(2):
                    if (pc, oc) not in m2:
                        continue
                    row = (pr * 2 + pc) * 32
                    for tr, kh in m2[(pr, orr)]:
                        for tc, kw in m2[(pc, oc)]:
                            col = tr * 12 + tc * 3
                            t2big = t2big.at[
                                orr * 3 + oc, row:row + 32, col:col + 3].set(
                                jnp.transpose(p['dt2w'][:, :, kh, kw], (1, 0)))
    t2b = jnp.tile(p['dt2b'], 16).reshape(1, 48)

    return [
        jnp.transpose(p['d1w'], (2, 3, 1, 0)).reshape(9, D, NH),
        p['d1b'].reshape(1, NH),
        dr1, db1, dr2, db2,
        t1big, t1b, t2big, t2b,
    ]


def kernel(x, params):
    p = params
    xim = _prep_xim(x)
    idx8, mind = _enc_call(xim, _enc_wts(p))
    idx_flat = idx8.reshape(NTOK)
    cb128 = jnp.pad(p['cb'], ((0, 0), (0, D)))
    q = _sc_gather_hist(cb128, idx_flat)
    q8 = q.reshape(B, NPIX, 2 * D)
    xr, vq, pp = _dec_call(q8, mind, idx8, _dec_wts(p))
    xr6 = xr.reshape(B, HQ, HQ, 4, 4, IC)
    x_recon = jnp.transpose(xr6, (0, 5, 1, 3, 2, 4)).reshape(B, IC, HW, HW)
    return x_recon, vq[0, 0], pp[0, 0], idx_flat
